# baseline (speedup 1.0000x reference)
import functools

import jax
import jax.numpy as jnp
from jax import lax
from jax.experimental import pallas as pl
from jax.experimental.pallas import tpu as pltpu
from jax.experimental.pallas import tpu_sc as plsc

_K = 2
_TAU = 1.0


def _gate_body(x_ref, w_ref, b_ref, ridx_ref, wts_ref):
    E = w_ref.shape[0]
    bB = x_ref.shape[0]
    logits = jax.lax.dot_general(
        x_ref[...], w_ref[...], (((1,), (1,)), ((), ())),
        preferred_element_type=jnp.float32,
    )
    logits = (logits + b_ref[...]) / _TAU

    m = jnp.max(logits, axis=1, keepdims=True)
    ex = jnp.exp(logits - m)
    g = ex / jnp.sum(ex, axis=1, keepdims=True)

    idx = jax.lax.broadcasted_iota(jnp.int32, logits.shape, 1)
    i1 = jnp.min(jnp.where(logits == m, idx, E), axis=1, keepdims=True)
    sel1 = idx == i1
    l2 = jnp.where(sel1, -jnp.inf, logits)
    m2 = jnp.max(l2, axis=1, keepdims=True)
    i2 = jnp.min(jnp.where(l2 == m2, idx, E), axis=1, keepdims=True)
    sel2 = idx == i2
    mask = sel1 | sel2

    adjusted = logits - jnp.log(jnp.maximum(_K * (g + 1e-10), 1e-10))
    am = jnp.where(mask, adjusted, -jnp.inf)
    amax = jnp.max(am, axis=1, keepdims=True)
    e2 = jnp.where(mask, jnp.exp(am - amax), 0.0)
    gs = e2 / jnp.sum(e2, axis=1, keepdims=True)

    w1 = jnp.sum(jnp.where(sel1, gs, 0.0), axis=1, keepdims=True)
    w2 = jnp.sum(jnp.where(sel2, gs, 0.0), axis=1, keepdims=True)

    row0 = pl.program_id(0) * bB * E
    gb = row0 + jax.lax.broadcasted_iota(jnp.int32, (bB, 1), 0) * E
    ridx_ref[...] = jnp.concatenate([gb + i1, gb + i2], axis=1)
    wts_ref[...] = jnp.concatenate(
        [jnp.broadcast_to(w1, (bB, 16)), jnp.broadcast_to(w2, (bB, 16))],
        axis=1,
    )


def _gate(x, W, b):
    B, D = x.shape
    E = W.shape[0]
    bB = 1024
    b2 = b.reshape(1, E).astype(jnp.float32)
    ridx, wts = pl.pallas_call(
        _gate_body,
        grid=(B // bB,),
        in_specs=[
            pl.BlockSpec((bB, D), lambda i: (i, 0)),
            pl.BlockSpec((E, D), lambda i: (0, 0)),
            pl.BlockSpec((1, E), lambda i: (0, 0)),
        ],
        out_specs=[
            pl.BlockSpec((bB, _K), lambda i: (i, 0)),
            pl.BlockSpec((bB, _K * 16), lambda i: (i, 0)),
        ],
        out_shape=[
            jax.ShapeDtypeStruct((B, _K), jnp.int32),
            jax.ShapeDtypeStruct((B, _K * 16), jnp.float32),
        ],
    )(x, W, b2)
    return ridx, wts


def _make_combine(B, D, E):
    info = plsc.get_sparse_core_info()
    NW = info.num_cores * info.num_subcores
    b_per_w = B // NW
    TB = 4
    nchunks = b_per_w // TB

    mesh = plsc.VectorSubcoreMesh(core_axis_name="c", subcore_axis_name="s")

    @functools.partial(
        pl.kernel,
        mesh=mesh,
        out_type=jax.ShapeDtypeStruct((B, D), jnp.float32),
        scratch_types=[
            pltpu.VMEM((_K * b_per_w,), jnp.int32),
            pltpu.VMEM((_K * b_per_w, 16), jnp.float32),
            pltpu.VMEM((4, _K * TB, D), jnp.float32),
            pltpu.VMEM((2, TB, D), jnp.float32),
            pltpu.SemaphoreType.DMA,
            pltpu.SemaphoreType.DMA,
            pltpu.SemaphoreType.DMA,
            pltpu.SemaphoreType.DMA,
            pltpu.SemaphoreType.DMA,
            pltpu.SemaphoreType.DMA,
        ],
    )
    def combine(table_hbm, idx_hbm, w_hbm, out_hbm,
                idx_v, w_v, buf, obuf, g0, g1, g2, g3, o0, o1):
        wid = lax.axis_index("s") * info.num_cores + lax.axis_index("c")
        base = wid * b_per_w
        pltpu.sync_copy(idx_hbm.at[pl.ds(base * _K, _K * b_per_w)], idx_v)
        pltpu.sync_copy(w_hbm.at[pl.ds(base * _K, _K * b_per_w), :], w_v)

        NG = 4
        NO = 2
        gsems = (g0, g1, g2, g3)
        osems = (o0, o1)

        def gather_copy(c, slot):
            return pltpu.make_async_copy(
                table_hbm.at[idx_v.at[pl.ds(c * _K * TB, _K * TB)]],
                buf.at[slot],
                gsems[slot],
            )

        def out_copy(c, slot):
            return pltpu.make_async_copy(
                obuf.at[slot],
                out_hbm.at[pl.ds(base + c * TB, TB)],
                osems[slot],
            )

        def compute_chunk(c, gslot, oslot):
            ws = []
            for t in range(TB):
                tok = c * TB + t
                ws.append((w_v[_K * tok, :], w_v[_K * tok + 1, :]))

            def jbody(j, ws=ws, gslot=gslot, oslot=oslot):
                o = j * 16
                for t in range(TB):
                    w1, w2 = ws[t]
                    a = buf[gslot, _K * t, pl.ds(o, 16)]
                    bvec = buf[gslot, _K * t + 1, pl.ds(o, 16)]
                    obuf[oslot, t, pl.ds(o, 16)] = w1 * a + w2 * bvec

            plsc.parallel_loop(0, D // 16, unroll=8)(jbody)

        gather_copy(0, 0).start()
        gather_copy(1, 1).start()
        gather_copy(2, 2).start()

        @pl.loop(0, nchunks, step=NG)
        def _chunks(cc):
            for gslot in range(NG):
                c = cc + gslot
                oslot = gslot % NO

                @pl.when(c + 3 < nchunks)
                def _():
                    gather_copy(c + 3, (gslot + 3) % NG).start()

                gather_copy(c, gslot).wait()

                @pl.when(c >= NO)
                def _():
                    out_copy(c - NO, oslot).wait()

                compute_chunk(c, gslot, oslot)
                out_copy(c, oslot).start()

        for c in range(nchunks - NO, nchunks):
            out_copy(c, c % NO).wait()

    return combine


def kernel(h, x, W, b):
    B, D, E = h.shape
    table = jnp.swapaxes(h, 1, 2).reshape(B * E, D)
    ridx, wts = _gate(x, W, b)
    combine = _make_combine(B, D, E)
    y = combine(table, ridx.reshape(B * _K), wts.reshape(B * _K, 16))
    return y


# speedup vs baseline: 1.0689x; 1.0689x over previous
import functools

import jax
import jax.numpy as jnp
from jax import lax
from jax.experimental import pallas as pl
from jax.experimental.pallas import tpu as pltpu
from jax.experimental.pallas import tpu_sc as plsc

_K = 2
_TAU = 1.0


def _gate_body(x_ref, w_ref, b_ref, ridx_ref, wts_ref):
    E = w_ref.shape[0]
    bB = x_ref.shape[0]
    logits = jax.lax.dot_general(
        x_ref[...], w_ref[...], (((1,), (1,)), ((), ())),
        preferred_element_type=jnp.float32,
    )
    logits = (logits + b_ref[...]) / _TAU

    m = jnp.max(logits, axis=1, keepdims=True)
    ex = jnp.exp(logits - m)
    g = ex / jnp.sum(ex, axis=1, keepdims=True)

    idx = jax.lax.broadcasted_iota(jnp.int32, logits.shape, 1)
    i1 = jnp.min(jnp.where(logits == m, idx, E), axis=1, keepdims=True)
    sel1 = idx == i1
    l2 = jnp.where(sel1, -jnp.inf, logits)
    m2 = jnp.max(l2, axis=1, keepdims=True)
    i2 = jnp.min(jnp.where(l2 == m2, idx, E), axis=1, keepdims=True)
    sel2 = idx == i2
    mask = sel1 | sel2

    adjusted = logits - jnp.log(jnp.maximum(_K * (g + 1e-10), 1e-10))
    am = jnp.where(mask, adjusted, -jnp.inf)
    amax = jnp.max(am, axis=1, keepdims=True)
    e2 = jnp.where(mask, jnp.exp(am - amax), 0.0)
    gs = e2 / jnp.sum(e2, axis=1, keepdims=True)

    w1 = jnp.sum(jnp.where(sel1, gs, 0.0), axis=1, keepdims=True)
    w2 = jnp.sum(jnp.where(sel2, gs, 0.0), axis=1, keepdims=True)

    row0 = pl.program_id(0) * bB * E
    gb = row0 + jax.lax.broadcasted_iota(jnp.int32, (bB, 1), 0) * E
    ridx_ref[...] = jnp.concatenate([gb + i1, gb + i2], axis=1)
    wts_ref[...] = jnp.concatenate(
        [jnp.broadcast_to(w1, (bB, 16)), jnp.broadcast_to(w2, (bB, 16))],
        axis=1,
    )


def _gate(x, W, b):
    B, D = x.shape
    E = W.shape[0]
    bB = 1024
    b2 = b.reshape(1, E).astype(jnp.float32)
    ridx, wts = pl.pallas_call(
        _gate_body,
        grid=(B // bB,),
        in_specs=[
            pl.BlockSpec((bB, D), lambda i: (i, 0)),
            pl.BlockSpec((E, D), lambda i: (0, 0)),
            pl.BlockSpec((1, E), lambda i: (0, 0)),
        ],
        out_specs=[
            pl.BlockSpec((bB, _K), lambda i: (i, 0)),
            pl.BlockSpec((bB, _K * 16), lambda i: (i, 0)),
        ],
        out_shape=[
            jax.ShapeDtypeStruct((B, _K), jnp.int32),
            jax.ShapeDtypeStruct((B, _K * 16), jnp.float32),
        ],
    )(x, W, b2)
    return ridx, wts


def _make_combine(B, D, E):
    info = plsc.get_sparse_core_info()
    NW = info.num_cores * info.num_subcores
    b_per_w = B // NW
    TB = 4
    nchunks = b_per_w // TB

    mesh = plsc.VectorSubcoreMesh(core_axis_name="c", subcore_axis_name="s")

    @functools.partial(
        pl.kernel,
        mesh=mesh,
        out_type=jax.ShapeDtypeStruct((B, D), jnp.float32),
        scratch_types=[
            pltpu.VMEM((_K * b_per_w,), jnp.int32),
            pltpu.VMEM((b_per_w, _K * 16), jnp.float32),
            pltpu.VMEM((4, _K * TB, D), jnp.float32),
            pltpu.VMEM((2, TB, D), jnp.float32),
            pltpu.SemaphoreType.DMA,
            pltpu.SemaphoreType.DMA,
            pltpu.SemaphoreType.DMA,
            pltpu.SemaphoreType.DMA,
            pltpu.SemaphoreType.DMA,
            pltpu.SemaphoreType.DMA,
        ],
    )
    def combine(table_hbm, idx_hbm, w_hbm, out_hbm,
                idx_v, w_v, buf, obuf, g0, g1, g2, g3, o0, o1):
        wid = lax.axis_index("s") * info.num_cores + lax.axis_index("c")
        base = wid * b_per_w
        pltpu.sync_copy(idx_hbm.at[pl.ds(base * _K, _K * b_per_w)], idx_v)
        pltpu.sync_copy(w_hbm.at[pl.ds(base, b_per_w), :], w_v)

        NG = 4
        NO = 2
        gsems = (g0, g1, g2, g3)
        osems = (o0, o1)

        def gather_copy(c, slot):
            return pltpu.make_async_copy(
                table_hbm.at[idx_v.at[pl.ds(c * _K * TB, _K * TB)]],
                buf.at[slot],
                gsems[slot],
            )

        def out_copy(c, slot):
            return pltpu.make_async_copy(
                obuf.at[slot],
                out_hbm.at[pl.ds(base + c * TB, TB)],
                osems[slot],
            )

        def compute_chunk(c, gslot, oslot):
            ws = []
            for t in range(TB):
                tok = c * TB + t
                ws.append((w_v[tok, pl.ds(0, 16)], w_v[tok, pl.ds(16, 16)]))

            def jbody(j, ws=ws, gslot=gslot, oslot=oslot):
                o = j * 16
                for t in range(TB):
                    w1, w2 = ws[t]
                    a = buf[gslot, _K * t, pl.ds(o, 16)]
                    bvec = buf[gslot, _K * t + 1, pl.ds(o, 16)]
                    obuf[oslot, t, pl.ds(o, 16)] = w1 * a + w2 * bvec

            plsc.parallel_loop(0, D // 16, unroll=8)(jbody)

        gather_copy(0, 0).start()
        gather_copy(1, 1).start()
        gather_copy(2, 2).start()

        @pl.loop(0, nchunks, step=NG)
        def _chunks(cc):
            for gslot in range(NG):
                c = cc + gslot
                oslot = gslot % NO

                @pl.when(c + 3 < nchunks)
                def _():
                    gather_copy(c + 3, (gslot + 3) % NG).start()

                gather_copy(c, gslot).wait()

                @pl.when(c >= NO)
                def _():
                    out_copy(c - NO, oslot).wait()

                compute_chunk(c, gslot, oslot)
                out_copy(c, oslot).start()

        for c in range(nchunks - NO, nchunks):
            out_copy(c, c % NO).wait()

    return combine


def kernel(h, x, W, b):
    B, D, E = h.shape
    table = jnp.swapaxes(h, 1, 2).reshape(B * E, D)
    ridx, wts = _gate(x, W, b)
    combine = _make_combine(B, D, E)
    y = combine(table, ridx.reshape(B * _K), wts)
    return y
